# Initial kernel scaffold; baseline (speedup 1.0000x reference)
import jax
import jax.numpy as jnp
from jax.experimental import pallas as pl


def kernel(x, edge_index, edge_attr, u, batch, W1, b1, W2, b2):
    raise NotImplementedError("write your pallas kernel here")


# speedup vs baseline: 2.0673x; 2.0673x over previous
import jax
import jax.numpy as jnp
from jax import lax
from jax.experimental import pallas as pl
from jax.experimental.pallas import tpu as pltpu
from jax.experimental.pallas import tpu_sc as plsc

NC = 2
NS = 16
NW = NC * NS
L = 16
CH = 128


def _seg_sum_sc(num_chunks, last_lo, g, d):
  q, big = divmod(num_chunks, NW)
  nj = d // L

  def body(x_hbm, idx_hbm, acc_out, xbuf, acc_v, idx_v):
    c = lax.axis_index("c")
    s = lax.axis_index("s")
    w = c * NS + s

    start = jnp.where(w < big, w * (q + 1), big * (q + 1) + (w - big) * q)
    cnt = jnp.where(w < big, q + 1, q)

    zv = jnp.zeros((L,), jnp.float32)

    def zero_step(i, _):
      acc_v[pl.ds(i * L, L)] = zv
      return 0

    lax.fori_loop(0, g * d // L, zero_step, 0)

    lane = lax.iota(jnp.int32, L)

    def flush(prev_v, carry):
      for j in range(nj):
        plsc.store_scatter(acc_v, [prev_v * d + j * L + lane], carry[j])

    def chunk_step(k, state):
      ch = start + k
      pltpu.sync_copy(idx_hbm.at[ch], idx_v)
      base = jnp.where(ch == num_chunks - 1, (num_chunks - 1) * CH - last_lo,
                       ch * CH)
      pltpu.sync_copy(x_hbm.at[pl.ds(base * d, CH * d)], xbuf)
      lo = jnp.where(ch == num_chunks - 1, last_lo, 0)

      def row_step(i, st):
        prev_v = st[0]
        carry = st[1:]
        seg_v = plsc.load_gather(idx_v, [jnp.full((L,), 0, jnp.int32) + i])
        new_run = jnp.any(seg_v != prev_v)

        @pl.when(new_run & jnp.any(prev_v >= 0))
        def _():
          flush(prev_v, carry)

        new_carry = tuple(
            jnp.where(new_run, xbuf[pl.ds(i * d + j * L, L)],
                      carry[j] + xbuf[pl.ds(i * d + j * L, L)])
            for j in range(nj))
        return (seg_v,) + new_carry

      return lax.fori_loop(lo, CH, row_step, state)

    init = (jnp.full((L,), -1, jnp.int32),) + tuple(
        jnp.zeros((L,), jnp.float32) for _ in range(nj))
    final = lax.fori_loop(0, cnt, chunk_step, init)
    prev_v = final[0]

    @pl.when(jnp.any(prev_v >= 0))
    def _():
      flush(prev_v, final[1:])

    pltpu.sync_copy(acc_v, acc_out.at[pl.ds(w * g * d, g * d)])

  return pl.kernel(
      body,
      out_type=jax.ShapeDtypeStruct((NW * g * d,), jnp.float32),
      mesh=plsc.VectorSubcoreMesh(core_axis_name="c", subcore_axis_name="s"),
      compiler_params=pltpu.CompilerParams(needs_layout_passes=False),
      scratch_types=[
          pltpu.VMEM((CH * d,), jnp.float32),
          pltpu.VMEM((g * d,), jnp.float32),
          pltpu.VMEM((CH,), jnp.int32),
      ],
  )


def _mlp_body(acc_ref, idx_ref, u_ref, w1u_ref, w1m_ref, b1_ref, w2_ref,
              b2_ref, o_ref):
  g = u_ref.shape[0]

  def red_step(i, carry):
    off = pl.multiple_of(i * g, 8)
    return carry + acc_ref[pl.ds(off, g), :]

  sums = lax.fori_loop(1, NW, red_step, acc_ref[pl.ds(0, g), :])

  segs = lax.broadcasted_iota(jnp.int32, (g, 1), 0)

  def cnt_step(i, carry):
    row = idx_ref[pl.ds(i, 1), :]
    eq = (row == segs).astype(jnp.float32)
    return carry + jnp.sum(eq, axis=1, keepdims=True)

  cnt = lax.fori_loop(0, idx_ref.shape[0], cnt_step,
                      jnp.zeros((g, 1), jnp.float32))

  mean = sums / jnp.maximum(cnt, 1.0)
  pre = (jnp.dot(u_ref[...], w1u_ref[...], preferred_element_type=jnp.float32)
         + jnp.dot(mean, w1m_ref[...], preferred_element_type=jnp.float32)
         + b1_ref[...])
  h = jnp.maximum(pre, 0.0)
  o_ref[...] = (jnp.dot(h, w2_ref[...], preferred_element_type=jnp.float32)
                + b2_ref[...])


@jax.jit
def kernel(x, edge_index, edge_attr, u, batch, W1, b1, W2, b2):
  del edge_index, edge_attr
  n, d = x.shape
  g, du = u.shape
  num_chunks = (n + CH - 1) // CH
  rem = n - (num_chunks - 1) * CH
  last_lo = CH - rem

  dummy = g

  batch32 = batch.astype(jnp.int32)
  if last_lo:
    idx = jnp.concatenate([
        batch32[: (num_chunks - 1) * CH],
        jnp.full((last_lo,), dummy, jnp.int32),
        batch32[(num_chunks - 1) * CH:],
    ])
  else:
    idx = batch32
  idx = idx.reshape(num_chunks, CH)

  acc = _seg_sum_sc(num_chunks, last_lo, g, d)(x.reshape(-1), idx)

  rows_pad = -(-num_chunks // 8) * 8
  idx_pad = jnp.concatenate(
      [idx, jnp.full((rows_pad - num_chunks, CH), dummy, jnp.int32)])

  return pl.pallas_call(
      _mlp_body,
      out_shape=jax.ShapeDtypeStruct((g, W2.shape[1]), jnp.float32),
  )(acc.reshape(NW * g, d), idx_pad, u, W1[:du], W1[du:], b1.reshape(1, -1),
    W2, b2.reshape(1, -1))


# speedup vs baseline: 3.3361x; 1.6137x over previous
import jax
import jax.numpy as jnp
from jax import lax
from jax.experimental import pallas as pl
from jax.experimental.pallas import tpu as pltpu
from jax.experimental.pallas import tpu_sc as plsc

NC = 2
NS = 16
NW = NC * NS
L = 16
CH = 128


def _seg_sum_sc(n, g, d):
  assert n % 8 == 0 and d % L == 0
  oct_total = n // 8
  ob, oe = divmod(oct_total, NW)
  max_range = 8 * (ob + 1 if oe else ob)
  nwin = -(-max_range // CH)
  assert nwin >= 2 and max_range >= CH
  nj = d // L

  def body(x_hbm, b_hbm, acc_out, xb0, xb1, iv0, iv1, acc_v, *sems):
    c = lax.axis_index("c")
    s = lax.axis_index("s")
    w = c * NS + s

    a0 = 8 * (w * ob + jnp.minimum(w, oe))
    rng = 8 * (ob + jnp.where(w < oe, 1, 0))

    xbufs = [xb0, xb1]
    ibufs = [iv0, iv1]

    def win_base(k):
      return jnp.minimum(a0 + k * CH, a0 + rng - CH)

    def make_copies(k):
      b = win_base(k)
      cx = pltpu.make_async_copy(x_hbm.at[pl.ds(b, CH)], xbufs[k % 2],
                                 sems[2 * k])
      ci = pltpu.make_async_copy(b_hbm.at[pl.ds(b, CH)], ibufs[k % 2],
                                 sems[2 * k + 1])
      return cx, ci

    copies = [make_copies(k) for k in range(nwin)]
    for k in range(min(2, nwin)):
      copies[k][0].start()
      copies[k][1].start()

    zv = jnp.zeros((L,), jnp.float32)

    def zero_step(i, _):
      for j in range(nj):
        acc_v[i, pl.ds(j * L, L)] = zv
      return 0

    lax.fori_loop(0, g, zero_step, 0)

    lane = lax.iota(jnp.int32, L)

    def flush(prev_v, carry):
      for j in range(nj):
        plsc.store_scatter(acc_v, [prev_v, lane + j * L], carry[j])

    def process(k, state):
      xbuf = xbufs[k % 2]
      ibuf = ibufs[k % 2]
      lo = jnp.maximum((k + 1) * CH - rng, 0)

      def row_step(i, st):
        prev_v = st[0]
        carry = st[1:]
        seg_v = plsc.load_gather(ibuf, [jnp.full((L,), 0, jnp.int32) + i])
        new_run = jnp.any(seg_v != prev_v)

        @pl.when(new_run & jnp.any(prev_v >= 0))
        def _():
          flush(prev_v, carry)

        new_carry = tuple(
            jnp.where(new_run, xbuf[i, pl.ds(j * L, L)],
                      carry[j] + xbuf[i, pl.ds(j * L, L)])
            for j in range(nj))
        return (seg_v,) + new_carry

      return lax.fori_loop(lo, CH, row_step, state)

    state = (jnp.full((L,), -1, jnp.int32),) + tuple(
        jnp.zeros((L,), jnp.float32) for _ in range(nj))
    for k in range(nwin):
      copies[k][0].wait()
      copies[k][1].wait()
      state = process(k, state)
      if k + 2 < nwin:
        copies[k + 2][0].start()
        copies[k + 2][1].start()

    @pl.when(jnp.any(state[0] >= 0))
    def _():
      flush(state[0], state[1:])

    pltpu.sync_copy(acc_v, acc_out.at[pl.ds(w * g, g)])

  return pl.kernel(
      body,
      out_type=jax.ShapeDtypeStruct((NW * g, d), jnp.float32),
      mesh=plsc.VectorSubcoreMesh(core_axis_name="c", subcore_axis_name="s"),
      compiler_params=pltpu.CompilerParams(needs_layout_passes=False),
      scratch_types=[
          pltpu.VMEM((CH, d), jnp.float32),
          pltpu.VMEM((CH, d), jnp.float32),
          pltpu.VMEM((CH,), jnp.int32),
          pltpu.VMEM((CH,), jnp.int32),
          pltpu.VMEM((g, d), jnp.float32),
      ] + [pltpu.SemaphoreType.DMA] * (2 * nwin),
  )


def _mlp_body(acc_ref, idx_ref, u_ref, w1u_ref, w1m_ref, b1_ref, w2_ref,
              b2_ref, o_ref):
  g = u_ref.shape[0]

  def red_step(i, carry):
    off = pl.multiple_of(i * g, 8)
    return carry + acc_ref[pl.ds(off, g), :]

  sums = lax.fori_loop(1, NW, red_step, acc_ref[pl.ds(0, g), :])

  segs = lax.broadcasted_iota(jnp.int32, (g, 1), 0)

  def cnt_step(i, carry):
    row = idx_ref[pl.ds(i, 1), :]
    eq = (row == segs).astype(jnp.float32)
    return carry + jnp.sum(eq, axis=1, keepdims=True)

  cnt = lax.fori_loop(0, idx_ref.shape[0], cnt_step,
                      jnp.zeros((g, 1), jnp.float32))

  mean = sums / jnp.maximum(cnt, 1.0)
  pre = (jnp.dot(u_ref[...], w1u_ref[...], preferred_element_type=jnp.float32)
         + jnp.dot(mean, w1m_ref[...], preferred_element_type=jnp.float32)
         + b1_ref[...])
  h = jnp.maximum(pre, 0.0)
  o_ref[...] = (jnp.dot(h, w2_ref[...], preferred_element_type=jnp.float32)
                + b2_ref[...])


@jax.jit
def kernel(x, edge_index, edge_attr, u, batch, W1, b1, W2, b2):
  del edge_index, edge_attr
  n, d = x.shape
  g, du = u.shape

  batch32 = batch.astype(jnp.int32)
  acc = _seg_sum_sc(n, g, d)(x, batch32)

  n_pad = -(-n // (8 * CH)) * (8 * CH)
  idx_pad = jnp.concatenate(
      [batch32, jnp.full((n_pad - n,), g, jnp.int32)]).reshape(-1, CH)

  return pl.pallas_call(
      _mlp_body,
      out_shape=jax.ShapeDtypeStruct((g, W2.shape[1]), jnp.float32),
  )(acc, idx_pad, u, W1[:du], W1[du:], b1.reshape(1, -1), W2,
    b2.reshape(1, -1))


# speedup vs baseline: 3.8816x; 1.1635x over previous
import jax
import jax.numpy as jnp
from jax import lax
from jax.experimental import pallas as pl
from jax.experimental.pallas import tpu as pltpu
from jax.experimental.pallas import tpu_sc as plsc

NC = 2
NS = 16
NW = NC * NS
L = 16
CH = 128
UNROLL = 4


def _seg_sum_sc(n, g, d):
  assert n % 8 == 0 and d % L == 0
  oct_total = n // 8
  ob, oe = divmod(oct_total, NW)
  max_range = 8 * (ob + 1 if oe else ob)
  nwin = -(-max_range // CH)
  assert nwin >= 2 and max_range >= CH
  nj = d // L

  def body(x_hbm, b_hbm, acc_out, xb0, xb1, iv0, iv1, acc_v, *sems):
    c = lax.axis_index("c")
    s = lax.axis_index("s")
    w = c * NS + s

    a0 = 8 * (w * ob + jnp.minimum(w, oe))
    rng = 8 * (ob + jnp.where(w < oe, 1, 0))

    xbufs = [xb0, xb1]
    ibufs = [iv0, iv1]

    def win_base(k):
      return jnp.minimum(a0 + k * CH, a0 + rng - CH)

    def make_copies(k):
      b = win_base(k)
      cx = pltpu.make_async_copy(x_hbm.at[pl.ds(b, CH)], xbufs[k % 2],
                                 sems[2 * k])
      ci = pltpu.make_async_copy(b_hbm.at[pl.ds(b, CH)], ibufs[k % 2],
                                 sems[2 * k + 1])
      return cx, ci

    copies = [make_copies(k) for k in range(nwin)]
    for k in range(min(2, nwin)):
      copies[k][0].start()
      copies[k][1].start()

    zv = jnp.zeros((L,), jnp.float32)

    def zero_step(i, _):
      for j in range(nj):
        acc_v[i, pl.ds(j * L, L)] = zv
      return 0

    lax.fori_loop(0, g, zero_step, 0)

    lane = lax.iota(jnp.int32, L)

    def flush(prev_v, carry):
      for j in range(nj):
        plsc.store_scatter(acc_v, [prev_v, lane + j * L], carry[j])

    def process(k, state):
      xbuf = xbufs[k % 2]
      ibuf = ibufs[k % 2]
      lo = jnp.maximum((k + 1) * CH - rng, 0)

      def one_row(i, st):
        prev_v = st[0]
        carry = st[1:]
        seg_v = plsc.load_gather(ibuf, [jnp.full((L,), 0, jnp.int32) + i])
        new_run = jnp.any(seg_v != prev_v)

        @pl.when(new_run & jnp.any(prev_v >= 0))
        def _():
          flush(prev_v, carry)

        new_carry = tuple(
            jnp.where(new_run, xbuf[i, pl.ds(j * L, L)],
                      carry[j] + xbuf[i, pl.ds(j * L, L)])
            for j in range(nj))
        return (seg_v,) + new_carry

      def head_row(i, st):
        return one_row(i, st)

      hi0 = jnp.minimum((lo + (UNROLL - 1)) // UNROLL * UNROLL, CH)
      state1 = lax.fori_loop(lo, hi0, head_row, state)

      def blk_step(b, st):
        i0 = b * UNROLL
        for r in range(UNROLL):
          st = one_row(i0 + r, st)
        return st

      return lax.fori_loop(hi0 // UNROLL, CH // UNROLL, blk_step, state1)

    state = (jnp.full((L,), -1, jnp.int32),) + tuple(
        jnp.zeros((L,), jnp.float32) for _ in range(nj))
    for k in range(nwin):
      copies[k][0].wait()
      copies[k][1].wait()
      state = process(k, state)
      if k + 2 < nwin:
        copies[k + 2][0].start()
        copies[k + 2][1].start()

    @pl.when(jnp.any(state[0] >= 0))
    def _():
      flush(state[0], state[1:])

    pltpu.sync_copy(acc_v, acc_out.at[pl.ds(w * g, g)])

  return pl.kernel(
      body,
      out_type=jax.ShapeDtypeStruct((NW * g, d), jnp.float32),
      mesh=plsc.VectorSubcoreMesh(core_axis_name="c", subcore_axis_name="s"),
      compiler_params=pltpu.CompilerParams(needs_layout_passes=False),
      scratch_types=[
          pltpu.VMEM((CH, d), jnp.float32),
          pltpu.VMEM((CH, d), jnp.float32),
          pltpu.VMEM((CH,), jnp.int32),
          pltpu.VMEM((CH,), jnp.int32),
          pltpu.VMEM((g, d), jnp.float32),
      ] + [pltpu.SemaphoreType.DMA] * (2 * nwin),
  )


def _mlp_body(acc_ref, idx_ref, u_ref, w1u_ref, w1m_ref, b1_ref, w2_ref,
              b2_ref, o_ref):
  g = u_ref.shape[0]

  def red_step(i, carry):
    off = pl.multiple_of(i * g, 8)
    return carry + acc_ref[pl.ds(off, g), :]

  sums = lax.fori_loop(1, NW, red_step, acc_ref[pl.ds(0, g), :])

  segs = lax.broadcasted_iota(jnp.int32, (g, 1), 0)
  cols = idx_ref.shape[1]

  def cnt_step(i, carry):
    row = idx_ref[pl.ds(i, 1), :]
    return carry + (row == segs).astype(jnp.float32)

  cnt2d = lax.fori_loop(0, idx_ref.shape[0], cnt_step,
                        jnp.zeros((g, cols), jnp.float32))
  cnt = jnp.dot(cnt2d, jnp.ones((cols, 1), jnp.float32),
                preferred_element_type=jnp.float32)

  mean = sums / jnp.maximum(cnt, 1.0)
  pre = (jnp.dot(u_ref[...], w1u_ref[...], preferred_element_type=jnp.float32)
         + jnp.dot(mean, w1m_ref[...], preferred_element_type=jnp.float32)
         + b1_ref[...])
  h = jnp.maximum(pre, 0.0)
  o_ref[...] = (jnp.dot(h, w2_ref[...], preferred_element_type=jnp.float32)
                + b2_ref[...])


@jax.jit
def kernel(x, edge_index, edge_attr, u, batch, W1, b1, W2, b2):
  del edge_index, edge_attr
  n, d = x.shape
  g, du = u.shape

  batch32 = batch.astype(jnp.int32)
  acc = _seg_sum_sc(n, g, d)(x, batch32)

  n_pad = -(-n // (8 * CH)) * (8 * CH)
  idx_pad = jnp.concatenate(
      [batch32, jnp.full((n_pad - n,), g, jnp.int32)]).reshape(-1, CH)

  return pl.pallas_call(
      _mlp_body,
      out_shape=jax.ShapeDtypeStruct((g, W2.shape[1]), jnp.float32),
  )(acc, idx_pad, u, W1[:du], W1[du:], b1.reshape(1, -1), W2,
    b2.reshape(1, -1))


# speedup vs baseline: 4.1643x; 1.0728x over previous
import jax
import jax.numpy as jnp
from jax import lax
from jax.experimental import pallas as pl
from jax.experimental.pallas import tpu as pltpu
from jax.experimental.pallas import tpu_sc as plsc

NC = 2
NS = 16
NW = NC * NS
L = 16
CH = 128
U = 8


def _seg_sum_sc(n, g, d):
  assert n % 8 == 0 and d % L == 0 and CH % U == 0
  oct_total = n // 8
  ob, oe = divmod(oct_total, NW)
  max_range = 8 * (ob + 1 if oe else ob)
  nwin = -(-max_range // CH)
  assert nwin >= 2 and max_range >= CH
  nj = d // L

  def body(x_hbm, b_hbm, acc_out, xb0, xb1, iv0, iv1, acc_v, sprev, sF, *sems):
    c = lax.axis_index("c")
    s = lax.axis_index("s")
    w = c * NS + s

    a0 = 8 * (w * ob + jnp.minimum(w, oe))
    rng = 8 * (ob + jnp.where(w < oe, 1, 0))

    xbufs = [xb0, xb1]
    ibufs = [iv0, iv1]

    def win_base(k):
      return jnp.minimum(a0 + k * CH, a0 + rng - CH)

    def make_copies(k):
      b = win_base(k)
      cx = pltpu.make_async_copy(x_hbm.at[pl.ds(b, CH)], xbufs[k % 2],
                                 sems[2 * k])
      ci = pltpu.make_async_copy(b_hbm.at[pl.ds(b, CH)], ibufs[k % 2],
                                 sems[2 * k + 1])
      return cx, ci

    copies = [make_copies(k) for k in range(nwin)]
    for k in range(min(2, nwin)):
      copies[k][0].start()
      copies[k][1].start()

    zv = jnp.zeros((L,), jnp.float32)

    def zero_step(i, _):
      for j in range(nj):
        acc_v[i, pl.ds(j * L, L)] = zv
      return 0

    lax.fori_loop(0, g, zero_step, 0)
    for j in range(nj):
      sF[pl.ds(j * L, L)] = zv
    sprev[...] = jnp.full((L,), -1, jnp.int32)

    lane = lax.iota(jnp.int32, L)
    blk_lane = jnp.minimum(lane, U - 1)

    def flush(prev_v, q):
      for j in range(nj):
        plsc.store_scatter(acc_v, [prev_v, lane + j * L],
                           q[j] - sF[pl.ds(j * L, L)])
        sF[pl.ds(j * L, L)] = q[j]

    def slow_row(xbuf, ibuf):
      def step(i, q):
        seg_v = plsc.load_gather(ibuf, [jnp.full((L,), 0, jnp.int32) + i])
        prev_v = sprev[...]

        @pl.when(jnp.any(seg_v != prev_v) & jnp.any(prev_v >= 0))
        def _():
          flush(prev_v, q)

        sprev[...] = seg_v
        return tuple(q[j] + xbuf[i, pl.ds(j * L, L)] for j in range(nj))

      return step

    def process(k, P):
      xbuf = xbufs[k % 2]
      ibuf = ibufs[k % 2]
      lo = jnp.maximum((k + 1) * CH - rng, 0)
      srow = slow_row(xbuf, ibuf)

      hi0 = jnp.minimum((lo + (U - 1)) // U * U, CH)
      P = lax.fori_loop(lo, hi0, srow, P)

      def blk_step(b, q):
        i0 = b * U
        ids = plsc.load_gather(ibuf, [blk_lane + i0])
        prev_v = sprev[...]

        @pl.when(jnp.any(ids != prev_v))
        def _():
          lax.fori_loop(i0, i0 + U, srow, q)

        for r in range(U):
          q = tuple(q[j] + xbuf[i0 + r, pl.ds(j * L, L)] for j in range(nj))
        return q

      return lax.fori_loop(hi0 // U, CH // U, blk_step, P)

    P = tuple(jnp.zeros((L,), jnp.float32) for _ in range(nj))
    for k in range(nwin):
      copies[k][0].wait()
      copies[k][1].wait()
      P = process(k, P)
      if k + 2 < nwin:
        copies[k + 2][0].start()
        copies[k + 2][1].start()

    prev_v = sprev[...]

    @pl.when(jnp.any(prev_v >= 0))
    def _():
      flush(prev_v, P)

    pltpu.sync_copy(acc_v, acc_out.at[pl.ds(w * g, g)])

  return pl.kernel(
      body,
      out_type=jax.ShapeDtypeStruct((NW * g, d), jnp.float32),
      mesh=plsc.VectorSubcoreMesh(core_axis_name="c", subcore_axis_name="s"),
      compiler_params=pltpu.CompilerParams(needs_layout_passes=False),
      scratch_types=[
          pltpu.VMEM((CH, d), jnp.float32),
          pltpu.VMEM((CH, d), jnp.float32),
          pltpu.VMEM((CH,), jnp.int32),
          pltpu.VMEM((CH,), jnp.int32),
          pltpu.VMEM((g, d), jnp.float32),
          pltpu.VMEM((L,), jnp.int32),
          pltpu.VMEM((d,), jnp.float32),
      ] + [pltpu.SemaphoreType.DMA] * (2 * nwin),
  )


def _mlp_body(acc_ref, idx_ref, u_ref, w1_ref, b1_ref, w2_ref, b2_ref, o_ref):
  g, du = u_ref.shape

  def red_step(i, carry):
    off = pl.multiple_of(i * g, 8)
    return carry + acc_ref[pl.ds(off, g), :]

  sums = lax.fori_loop(1, NW, red_step, acc_ref[pl.ds(0, g), :])

  segs = lax.broadcasted_iota(jnp.int32, (g, 1), 0)
  cols = idx_ref.shape[1]

  def cnt_step(i, carry):
    row = idx_ref[pl.ds(i, 1), :]
    return carry + (row == segs).astype(jnp.float32)

  cnt2d = lax.fori_loop(0, idx_ref.shape[0], cnt_step,
                        jnp.zeros((g, cols), jnp.float32))
  cnt = jnp.dot(cnt2d, jnp.ones((cols, 1), jnp.float32),
                preferred_element_type=jnp.float32)

  mean = sums / jnp.maximum(cnt, 1.0)
  pre = (jnp.dot(u_ref[...], w1_ref[pl.ds(0, du), :],
                 preferred_element_type=jnp.float32)
         + jnp.dot(mean, w1_ref[pl.ds(du, acc_ref.shape[1]), :],
                   preferred_element_type=jnp.float32)
         + b1_ref[...])
  h = jnp.maximum(pre, 0.0)
  o_ref[...] = (jnp.dot(h, w2_ref[...], preferred_element_type=jnp.float32)
                + b2_ref[...])


@jax.jit
def kernel(x, edge_index, edge_attr, u, batch, W1, b1, W2, b2):
  del edge_index, edge_attr
  n, d = x.shape
  g, du = u.shape

  batch32 = batch.astype(jnp.int32)
  acc = _seg_sum_sc(n, g, d)(x, batch32)

  n_pad = -(-n // (8 * CH)) * (8 * CH)
  idx_pad = jnp.concatenate(
      [batch32, jnp.full((n_pad - n,), g, jnp.int32)]).reshape(-1, CH)

  return pl.pallas_call(
      _mlp_body,
      out_shape=jax.ShapeDtypeStruct((g, W2.shape[1]), jnp.float32),
  )(acc, idx_pad, u, W1, b1.reshape(1, -1), W2, b2.reshape(1, -1))
